# baseline (speedup 1.0000x reference)
import jax
import jax.numpy as jnp
from jax import lax
from jax.experimental import pallas as pl
from jax.experimental.pallas import tpu as pltpu
from jax.experimental.pallas import tpu_sc as plsc

VOCAB = 1_000_000
DIM = 32
B, L = 16384, 50
STRIDE = 16
NSAMP = VOCAB // STRIDE
NPAD = 62504
PAD_VAL = 2**31 - 1

NC, NS = 2, 16
NW = NC * NS
ROWS_PER_W = B // NW
RCHUNK = 16
NCHUNK = ROWS_PER_W // RCHUNK


def _body(raw_hbm, keys2d_hbm, samp_hbm, table_hbm, out_hbm,
          samp_v, x_v, r_v, qrow_v, rows_v, emb_v, p_v, sem):
    wid = lax.axis_index("s") * NC + lax.axis_index("c")
    pltpu.sync_copy(samp_hbm, samp_v)

    lane = lax.broadcasted_iota(jnp.int32, (16,), 0)

    def chunk_body(k, carry):
        row_lo = wid * ROWS_PER_W + k * RCHUNK
        pltpu.sync_copy(raw_hbm.at[pl.ds(row_lo, RCHUNK)], x_v)

        for l in range(L):
            lv = jnp.full((16,), l, jnp.int32)
            x = plsc.load_gather(x_v, [lane, lv])
            q = jnp.zeros((16,), jnp.int32)
            ln = NSAMP
            while ln > 1:
                half = ln >> 1
                probe = plsc.load_gather(samp_v, [q + (half - 1)])
                q = jnp.where(probe < x, q + half, q)
                ln -= half
            probe = plsc.load_gather(samp_v, [q])
            q = jnp.where(probe < x, q + 1, q)
            r_v[l, :] = jnp.maximum(q - 1, 0)

        cps = [pltpu.async_copy(keys2d_hbm.at[r_v.at[l]], qrow_v.at[l], sem)
               for l in range(L)]
        for cp in cps:
            cp.wait()

        for l in range(L):
            lv = jnp.full((16,), l, jnp.int32)
            x = plsc.load_gather(x_v, [lane, lv])
            r = r_v[l, :]
            c = jnp.zeros((16,), jnp.int32)
            for s in (8, 4, 2, 1):
                probe = plsc.load_gather(qrow_v, [lv, lane, c + (s - 1)])
                c = jnp.where(probe < x, c + s, c)
            probe = plsc.load_gather(qrow_v, [lv, lane, c])
            c = jnp.where(probe < x, c + 1, c)
            idx = r * STRIDE + c
            keyval_in = plsc.load_gather(qrow_v, [lv, lane, jnp.minimum(c, 15)])
            keyval_out = plsc.load_gather(samp_v, [r + 1])
            keyval = jnp.where(c < STRIDE, keyval_in, keyval_out)
            valid = (idx < VOCAB) & (keyval == x)
            rows_v[l, :] = jnp.where(valid, idx + 1, 0)

        cps = [pltpu.async_copy(table_hbm.at[rows_v.at[l]], emb_v.at[l], sem)
               for l in range(L)]
        for cp in cps:
            cp.wait()

        def tr_body(l, carry2):
            lv = jnp.full((16,), l, jnp.int32)
            vals = [plsc.load_gather(emb_v, [lv, lane, jnp.full((16,), d, jnp.int32)])
                    for d in range(DIM)]
            for d in range(DIM):
                p_v[l, d, :] = vals[d]
            return carry2
        lax.fori_loop(0, L, tr_body, 0)
        pltpu.sync_copy(p_v, out_hbm.at[:, :, pl.ds(row_lo, RCHUNK)])
        return carry

    lax.fori_loop(0, NCHUNK, chunk_body, 0)


@jax.jit
def _lookup(raw_idx, keys2d, samp, table):
    mesh = plsc.VectorSubcoreMesh(core_axis_name="c", subcore_axis_name="s",
                                  num_cores=NC, num_subcores=NS)
    f = pl.kernel(
        _body,
        out_type=jax.ShapeDtypeStruct((L, DIM, B), jnp.float32),
        mesh=mesh,
        compiler_params=pltpu.CompilerParams(needs_layout_passes=False,
                                             use_tc_tiling_on_sc=False),
        scratch_types=[
            pltpu.VMEM((NPAD,), jnp.int32),
            pltpu.VMEM((RCHUNK, L), jnp.int32),
            pltpu.VMEM((L, RCHUNK), jnp.int32),
            pltpu.VMEM((L, RCHUNK, STRIDE), jnp.int32),
            pltpu.VMEM((L, RCHUNK), jnp.int32),
            pltpu.VMEM((L, RCHUNK, DIM), jnp.float32),
            pltpu.VMEM((L, DIM, RCHUNK), jnp.float32),
            pltpu.SemaphoreType.DMA,
        ],
    )
    return f(raw_idx, keys2d, samp, table)


def kernel(raw_idx, keys, table):
    keys2d = keys.reshape(NSAMP, STRIDE)
    samp = jnp.full((NPAD,), PAD_VAL, jnp.int32).at[:NSAMP].set(keys2d[:, 0])
    out_p = _lookup(raw_idx, keys2d, samp, table)
    return jnp.transpose(out_p, (2, 0, 1))


# speedup vs baseline: 1.0059x; 1.0059x over previous
import jax
import jax.numpy as jnp
from jax import lax
from jax.experimental import pallas as pl
from jax.experimental.pallas import tpu as pltpu
from jax.experimental.pallas import tpu_sc as plsc

VOCAB = 1_000_000
DIM = 32
B, L = 16384, 50
STRIDE = 16
NSAMP = VOCAB // STRIDE
NPAD = 62504
PAD_VAL = 2**31 - 1

NC, NS = 2, 16
NW = NC * NS
ROWS_PER_W = B // NW
RCHUNK = 16
NCHUNK = ROWS_PER_W // RCHUNK


def _body(raw_hbm, keys2d_hbm, samp_hbm, table_hbm, out_hbm,
          samp_v, x_v, r_v, qrow_v, rows_v, emb_v, p_v, sem):
    wid = lax.axis_index("s") * NC + lax.axis_index("c")
    pltpu.sync_copy(samp_hbm, samp_v)

    lane = lax.broadcasted_iota(jnp.int32, (16,), 0)

    def chunk_body(k, carry):
        row_lo = wid * ROWS_PER_W + k * RCHUNK
        pltpu.sync_copy(raw_hbm.at[pl.ds(row_lo, RCHUNK)], x_v)

        for l in range(L):
            lv = jnp.full((16,), l, jnp.int32)
            x = plsc.load_gather(x_v, [lane, lv])
            q = jnp.zeros((16,), jnp.int32)
            ln = NSAMP
            while ln > 1:
                half = ln >> 1
                probe = plsc.load_gather(samp_v, [q + (half - 1)])
                q = jnp.where(probe < x, q + half, q)
                ln -= half
            probe = plsc.load_gather(samp_v, [q])
            q = jnp.where(probe < x, q + 1, q)
            r_v[l, :] = jnp.maximum(q - 1, 0)

        cps = [pltpu.async_copy(keys2d_hbm.at[r_v.at[l]], qrow_v.at[l], sem)
               for l in range(L)]
        for cp in cps:
            cp.wait()

        for l in range(L):
            lv = jnp.full((16,), l, jnp.int32)
            x = plsc.load_gather(x_v, [lane, lv])
            r = r_v[l, :]
            c = jnp.zeros((16,), jnp.int32)
            for s in (8, 4, 2, 1):
                probe = plsc.load_gather(qrow_v, [lv, lane, c + (s - 1)])
                c = jnp.where(probe < x, c + s, c)
            probe = plsc.load_gather(qrow_v, [lv, lane, c])
            c = jnp.where(probe < x, c + 1, c)
            idx = r * STRIDE + c
            keyval_in = plsc.load_gather(qrow_v, [lv, lane, jnp.minimum(c, 15)])
            keyval_out = plsc.load_gather(samp_v, [r + 1])
            keyval = jnp.where(c < STRIDE, keyval_in, keyval_out)
            valid = (idx < VOCAB) & (keyval == x)
            rows_v[l, :] = jnp.where(valid, (idx + 1) * 4, 0)

        cps = [pltpu.async_copy(table_hbm.at[rows_v.at[l]], emb_v.at[l], sem)
               for l in range(L)]
        for cp in cps:
            cp.wait()

        def tr_body(l, carry2):
            lv = jnp.full((16,), l, jnp.int32)
            vals = [plsc.load_gather(emb_v, [lv, lane, jnp.full((16,), d, jnp.int32)])
                    for d in range(DIM)]
            for d in range(DIM):
                p_v[l, d, :] = vals[d]
            return carry2
        lax.fori_loop(0, L, tr_body, 0)
        pltpu.sync_copy(p_v, out_hbm.at[:, :, pl.ds(row_lo, RCHUNK)])
        return carry

    lax.fori_loop(0, NCHUNK, chunk_body, 0)


@jax.jit
def _lookup(raw_idx, keys2d, samp, table):
    mesh = plsc.VectorSubcoreMesh(core_axis_name="c", subcore_axis_name="s",
                                  num_cores=NC, num_subcores=NS)
    f = pl.kernel(
        _body,
        out_type=jax.ShapeDtypeStruct((L, DIM, B), jnp.float32),
        mesh=mesh,
        compiler_params=pltpu.CompilerParams(needs_layout_passes=False,
                                             use_tc_tiling_on_sc=False),
        scratch_types=[
            pltpu.VMEM((NPAD,), jnp.int32),
            pltpu.VMEM((RCHUNK, L), jnp.int32),
            pltpu.VMEM((L, RCHUNK), jnp.int32),
            pltpu.VMEM((L, RCHUNK, STRIDE), jnp.int32),
            pltpu.VMEM((L, RCHUNK), jnp.int32),
            pltpu.VMEM((L, RCHUNK, DIM), jnp.float32),
            pltpu.VMEM((L, DIM, RCHUNK), jnp.float32),
            pltpu.SemaphoreType.DMA,
        ],
    )
    return f(raw_idx, keys2d, samp, table)


def kernel(raw_idx, keys, table):
    keys2d = keys.reshape(NSAMP, STRIDE)
    samp = jnp.full((NPAD,), PAD_VAL, jnp.int32).at[:NSAMP].set(keys2d[:, 0])
    tview = jnp.pad(table, ((0, 7), (0, 96))).reshape(4 * (VOCAB + 8), DIM)
    out_p = _lookup(raw_idx, keys2d, samp, tview)
    return jnp.transpose(out_p, (2, 0, 1))


# speedup vs baseline: 1.1637x; 1.1569x over previous
import jax
import jax.numpy as jnp
from jax import lax
from jax.experimental import pallas as pl
from jax.experimental.pallas import tpu as pltpu
from jax.experimental.pallas import tpu_sc as plsc

VOCAB = 1_000_000
DIM = 32
B, L = 16384, 50
STRIDE = 16
NSAMP = VOCAB // STRIDE
NPAD = 62504
PAD_VAL = 2**31 - 1

NC, NS = 2, 16
NW = NC * NS
ROWS_PER_W = B // NW
RCHUNK = 16
NCHUNK = ROWS_PER_W // RCHUNK


def _search_body(raw_hbm, keys2d_hbm, samp_hbm, rows_hbm,
                 samp_v, x_v, r_v, qrow_v, rows_v, sem):
    wid = lax.axis_index("s") * NC + lax.axis_index("c")
    pltpu.sync_copy(samp_hbm, samp_v)

    lane = lax.broadcasted_iota(jnp.int32, (16,), 0)

    def chunk_body(k, carry):
        row_lo = wid * ROWS_PER_W + k * RCHUNK
        pltpu.sync_copy(raw_hbm.at[pl.ds(row_lo, RCHUNK)], x_v)

        for l in range(L):
            lv = jnp.full((16,), l, jnp.int32)
            x = plsc.load_gather(x_v, [lane, lv])
            q = jnp.zeros((16,), jnp.int32)
            ln = NSAMP
            while ln > 1:
                half = ln >> 1
                probe = plsc.load_gather(samp_v, [q + (half - 1)])
                q = jnp.where(probe < x, q + half, q)
                ln -= half
            probe = plsc.load_gather(samp_v, [q])
            q = jnp.where(probe < x, q + 1, q)
            r_v[l, :] = jnp.maximum(q - 1, 0)

        cps = [pltpu.async_copy(keys2d_hbm.at[r_v.at[l]], qrow_v.at[l], sem)
               for l in range(L)]
        for cp in cps:
            cp.wait()

        for l in range(L):
            lv = jnp.full((16,), l, jnp.int32)
            x = plsc.load_gather(x_v, [lane, lv])
            r = r_v[l, :]
            c = jnp.zeros((16,), jnp.int32)
            for s in (8, 4, 2, 1):
                probe = plsc.load_gather(qrow_v, [lv, lane, c + (s - 1)])
                c = jnp.where(probe < x, c + s, c)
            probe = plsc.load_gather(qrow_v, [lv, lane, c])
            c = jnp.where(probe < x, c + 1, c)
            idx = r * STRIDE + c
            keyval_in = plsc.load_gather(qrow_v, [lv, lane, jnp.minimum(c, 15)])
            keyval_out = plsc.load_gather(samp_v, [r + 1])
            keyval = jnp.where(c < STRIDE, keyval_in, keyval_out)
            valid = (idx < VOCAB) & (keyval == x)
            rows_v[l, :] = jnp.where(valid, idx + 1, 0)

        pltpu.sync_copy(rows_v, rows_hbm.at[:, pl.ds(row_lo, RCHUNK)])
        return carry

    lax.fori_loop(0, NCHUNK, chunk_body, 0)


def _gather_body(rows_hbm, table_hbm, out_hbm, rows_v, emb_v, p_v, sem):
    wid = lax.axis_index("s") * NC + lax.axis_index("c")
    lane = lax.broadcasted_iota(jnp.int32, (16,), 0)

    def chunk_body(k, carry):
        row_lo = wid * ROWS_PER_W + k * RCHUNK
        pltpu.sync_copy(rows_hbm.at[:, pl.ds(row_lo, RCHUNK)], rows_v)

        cps = [pltpu.async_copy(table_hbm.at[rows_v.at[l]], emb_v.at[l], sem)
               for l in range(L)]
        for cp in cps:
            cp.wait()

        def tr_body(l, carry2):
            lv = jnp.full((16,), l, jnp.int32)
            vals = [plsc.load_gather(emb_v, [lv, lane, jnp.full((16,), d, jnp.int32)])
                    for d in range(DIM)]
            for d in range(DIM):
                p_v[l, d, :] = vals[d]
            return carry2
        lax.fori_loop(0, L, tr_body, 0)
        pltpu.sync_copy(p_v, out_hbm.at[:, :, pl.ds(row_lo, RCHUNK)])
        return carry

    lax.fori_loop(0, NCHUNK, chunk_body, 0)


@jax.jit
def _lookup(raw_idx, keys2d, samp, table):
    mesh = plsc.VectorSubcoreMesh(core_axis_name="c", subcore_axis_name="s",
                                  num_cores=NC, num_subcores=NS)
    params = pltpu.CompilerParams(needs_layout_passes=False,
                                  use_tc_tiling_on_sc=False)
    k1 = pl.kernel(
        _search_body,
        out_type=jax.ShapeDtypeStruct((L, B), jnp.int32),
        mesh=mesh,
        compiler_params=params,
        scratch_types=[
            pltpu.VMEM((NPAD,), jnp.int32),
            pltpu.VMEM((RCHUNK, L), jnp.int32),
            pltpu.VMEM((L, RCHUNK), jnp.int32),
            pltpu.VMEM((L, RCHUNK, STRIDE), jnp.int32),
            pltpu.VMEM((L, RCHUNK), jnp.int32),
            pltpu.SemaphoreType.DMA,
        ],
    )
    rows_all = k1(raw_idx, keys2d, samp)
    k2 = pl.kernel(
        _gather_body,
        out_type=jax.ShapeDtypeStruct((L, DIM, B), jnp.float32),
        mesh=mesh,
        compiler_params=params,
        scratch_types=[
            pltpu.VMEM((L, RCHUNK), jnp.int32),
            pltpu.VMEM((L, RCHUNK, DIM), jnp.float32),
            pltpu.VMEM((L, DIM, RCHUNK), jnp.float32),
            pltpu.SemaphoreType.DMA,
        ],
    )
    return k2(rows_all, table)


def kernel(raw_idx, keys, table):
    keys2d = keys.reshape(NSAMP, STRIDE)
    samp = jnp.full((NPAD,), PAD_VAL, jnp.int32).at[:NSAMP].set(keys2d[:, 0])
    out_p = _lookup(raw_idx, keys2d, samp, table)
    return jnp.transpose(out_p, (2, 0, 1))


# speedup vs baseline: 1.2509x; 1.0749x over previous
import jax
import jax.numpy as jnp
from jax import lax
from jax.experimental import pallas as pl
from jax.experimental.pallas import tpu as pltpu
from jax.experimental.pallas import tpu_sc as plsc

VOCAB = 1_000_000
DIM = 32
B, L = 16384, 50
STRIDE = 16
NSAMP = VOCAB // STRIDE
NPAD = 62504
PAD_VAL = 2**31 - 1

NC, NS = 2, 16
NW = NC * NS
ROWS_PER_W = B // NW
RCHUNK = 16
NCHUNK = ROWS_PER_W // RCHUNK


def _search_body(raw_hbm, keys2d_hbm, samp_hbm, rows_hbm,
                 samp_v, x_v, r_v, qrow_v, rows_v, sem):
    wid = lax.axis_index("s") * NC + lax.axis_index("c")
    pltpu.sync_copy(samp_hbm, samp_v)

    lane = lax.broadcasted_iota(jnp.int32, (16,), 0)

    def chunk_body(k, carry):
        row_lo = wid * ROWS_PER_W + k * RCHUNK
        pltpu.sync_copy(raw_hbm.at[pl.ds(row_lo, RCHUNK)], x_v)

        cps = []
        for l in range(L):
            lv = jnp.full((16,), l, jnp.int32)
            x = plsc.load_gather(x_v, [lane, lv])
            q = jnp.zeros((16,), jnp.int32)
            ln = NSAMP
            while ln > 1:
                half = ln >> 1
                probe = plsc.load_gather(samp_v, [q + (half - 1)])
                q = jnp.where(probe < x, q + half, q)
                ln -= half
            probe = plsc.load_gather(samp_v, [q])
            q = jnp.where(probe < x, q + 1, q)
            r_v[l, :] = jnp.maximum(q - 1, 0)
            cps.append(pltpu.async_copy(keys2d_hbm.at[r_v.at[l]],
                                        qrow_v.at[l], sem))
        for cp in cps:
            cp.wait()

        for l in range(L):
            lv = jnp.full((16,), l, jnp.int32)
            x = plsc.load_gather(x_v, [lane, lv])
            r = r_v[l, :]
            c = jnp.zeros((16,), jnp.int32)
            for s in (8, 4, 2, 1):
                probe = plsc.load_gather(qrow_v, [lv, lane, c + (s - 1)])
                c = jnp.where(probe < x, c + s, c)
            probe = plsc.load_gather(qrow_v, [lv, lane, c])
            c = jnp.where(probe < x, c + 1, c)
            idx = r * STRIDE + c
            keyval_in = plsc.load_gather(qrow_v, [lv, lane, jnp.minimum(c, 15)])
            keyval_out = plsc.load_gather(samp_v, [r + 1])
            keyval = jnp.where(c < STRIDE, keyval_in, keyval_out)
            valid = (idx < VOCAB) & (keyval == x)
            rows_v[l, :] = jnp.where(valid, idx + 1, 0)

        pltpu.sync_copy(rows_v, rows_hbm.at[:, pl.ds(row_lo, RCHUNK)])
        return carry

    lax.fori_loop(0, NCHUNK, chunk_body, 0)


def _gather_body(rows_hbm, table_hbm, out_hbm,
                 rows_a, rows_b, emb_a, emb_b, p_a, p_b,
                 sem_a, sem_b, sem_w):
    wid = lax.axis_index("s") * NC + lax.axis_index("c")
    lane = lax.broadcasted_iota(jnp.int32, (16,), 0)

    def fetch(k, rows_v, emb_v, sem):
        row_lo = wid * ROWS_PER_W + k * RCHUNK
        pltpu.sync_copy(rows_hbm.at[:, pl.ds(row_lo, RCHUNK)], rows_v)
        return [pltpu.async_copy(table_hbm.at[rows_v.at[l]], emb_v.at[l], sem)
                for l in range(L)]

    def produce(k, emb_v, p_v, cps):
        row_lo = wid * ROWS_PER_W + k * RCHUNK
        for cp in cps:
            cp.wait()

        def tr_body(l, carry2):
            lv = jnp.full((16,), l, jnp.int32)
            vals = [plsc.load_gather(emb_v, [lv, lane, jnp.full((16,), d, jnp.int32)])
                    for d in range(DIM)]
            for d in range(DIM):
                p_v[l, d, :] = vals[d]
            return carry2
        lax.fori_loop(0, L, tr_body, 0)
        return pltpu.async_copy(p_v, out_hbm.at[:, :, pl.ds(row_lo, RCHUNK)],
                                sem_w)

    def pair_body(kk, carry):
        k0 = 2 * kk
        k1 = k0 + 1
        cps_a = fetch(k0, rows_a, emb_a, sem_a)
        cps_b = fetch(k1, rows_b, emb_b, sem_b)
        wa = produce(k0, emb_a, p_a, cps_a)
        wb = produce(k1, emb_b, p_b, cps_b)
        wa.wait()
        wb.wait()
        return carry

    lax.fori_loop(0, NCHUNK // 2, pair_body, 0)


@jax.jit
def _lookup(raw_idx, keys2d, samp, table):
    mesh = plsc.VectorSubcoreMesh(core_axis_name="c", subcore_axis_name="s",
                                  num_cores=NC, num_subcores=NS)
    params = pltpu.CompilerParams(needs_layout_passes=False,
                                  use_tc_tiling_on_sc=False)
    k1 = pl.kernel(
        _search_body,
        out_type=jax.ShapeDtypeStruct((L, B), jnp.int32),
        mesh=mesh,
        compiler_params=params,
        scratch_types=[
            pltpu.VMEM((NPAD,), jnp.int32),
            pltpu.VMEM((RCHUNK, L), jnp.int32),
            pltpu.VMEM((L, RCHUNK), jnp.int32),
            pltpu.VMEM((L, RCHUNK, STRIDE), jnp.int32),
            pltpu.VMEM((L, RCHUNK), jnp.int32),
            pltpu.SemaphoreType.DMA,
        ],
    )
    rows_all = k1(raw_idx, keys2d, samp)
    k2 = pl.kernel(
        _gather_body,
        out_type=jax.ShapeDtypeStruct((L, DIM, B), jnp.float32),
        mesh=mesh,
        compiler_params=params,
        scratch_types=[
            pltpu.VMEM((L, RCHUNK), jnp.int32),
            pltpu.VMEM((L, RCHUNK), jnp.int32),
            pltpu.VMEM((L, RCHUNK, DIM), jnp.float32),
            pltpu.VMEM((L, RCHUNK, DIM), jnp.float32),
            pltpu.VMEM((L, DIM, RCHUNK), jnp.float32),
            pltpu.VMEM((L, DIM, RCHUNK), jnp.float32),
            pltpu.SemaphoreType.DMA,
            pltpu.SemaphoreType.DMA,
            pltpu.SemaphoreType.DMA,
        ],
    )
    return k2(rows_all, table)


def kernel(raw_idx, keys, table):
    keys2d = keys.reshape(NSAMP, STRIDE)
    samp = jnp.full((NPAD,), PAD_VAL, jnp.int32).at[:NSAMP].set(keys2d[:, 0])
    out_p = _lookup(raw_idx, keys2d, samp, table)
    return jnp.transpose(out_p, (2, 0, 1))


# speedup vs baseline: 1.4233x; 1.1378x over previous
import jax
import jax.numpy as jnp
from jax import lax
from jax.experimental import pallas as pl
from jax.experimental.pallas import tpu as pltpu
from jax.experimental.pallas import tpu_sc as plsc

VOCAB = 1_000_000
DIM = 32
B, L = 16384, 50
STRIDE = 16
NSAMP = VOCAB // STRIDE
NPAD = 62504
PAD_VAL = 2**31 - 1

NC, NS = 2, 16
NW = NC * NS
ROWS_PER_W = B // NW
RCHUNK = 16
NCHUNK = ROWS_PER_W // RCHUNK


def _search_body(raw_hbm, keys2d_hbm, samp_hbm, rows_hbm,
                 samp_v, x_v, r_v, qrow_v, rows_v, sem):
    wid = lax.axis_index("s") * NC + lax.axis_index("c")
    pltpu.sync_copy(samp_hbm, samp_v)

    lane = lax.broadcasted_iota(jnp.int32, (16,), 0)

    def chunk_body(k, carry):
        row_lo = wid * ROWS_PER_W + k * RCHUNK
        pltpu.sync_copy(raw_hbm.at[pl.ds(row_lo, RCHUNK)], x_v)

        G = 5
        cps = []
        for l0 in range(0, L, G):
            ls = list(range(l0, l0 + G))
            lvs = [jnp.full((16,), l, jnp.int32) for l in ls]
            xs = [plsc.load_gather(x_v, [lane, lv]) for lv in lvs]
            qs = [jnp.zeros((16,), jnp.int32) for _ in ls]
            ln = NSAMP
            while ln > 1:
                half = ln >> 1
                probes = [plsc.load_gather(samp_v, [q + (half - 1)]) for q in qs]
                qs = [jnp.where(p < x, q + half, q)
                      for p, x, q in zip(probes, xs, qs)]
                ln -= half
            probes = [plsc.load_gather(samp_v, [q]) for q in qs]
            qs = [jnp.where(p < x, q + 1, q) for p, x, q in zip(probes, xs, qs)]
            for i, l in enumerate(ls):
                r_v[l, :] = jnp.maximum(qs[i] - 1, 0)
            for l in ls:
                cps.append(pltpu.async_copy(keys2d_hbm.at[r_v.at[l]],
                                            qrow_v.at[l], sem))
        for cp in cps:
            cp.wait()

        for l0 in range(0, L, G):
            ls = list(range(l0, l0 + G))
            lvs = [jnp.full((16,), l, jnp.int32) for l in ls]
            xs = [plsc.load_gather(x_v, [lane, lv]) for lv in lvs]
            rs = [r_v[l, :] for l in ls]
            cs = [jnp.zeros((16,), jnp.int32) for _ in ls]
            for s in (8, 4, 2, 1):
                probes = [plsc.load_gather(qrow_v, [lv, lane, c + (s - 1)])
                          for lv, c in zip(lvs, cs)]
                cs = [jnp.where(p < x, c + s, c)
                      for p, x, c in zip(probes, xs, cs)]
            probes = [plsc.load_gather(qrow_v, [lv, lane, c])
                      for lv, c in zip(lvs, cs)]
            cs = [jnp.where(p < x, c + 1, c) for p, x, c in zip(probes, xs, cs)]
            kv_ins = [plsc.load_gather(qrow_v, [lv, lane, jnp.minimum(c, 15)])
                      for lv, c in zip(lvs, cs)]
            kv_outs = [plsc.load_gather(samp_v, [r + 1]) for r in rs]
            for i, l in enumerate(ls):
                idx = rs[i] * STRIDE + cs[i]
                keyval = jnp.where(cs[i] < STRIDE, kv_ins[i], kv_outs[i])
                valid = (idx < VOCAB) & (keyval == xs[i])
                rows_v[l, :] = jnp.where(valid, idx + 1, 0)

        pltpu.sync_copy(rows_v, rows_hbm.at[:, pl.ds(row_lo, RCHUNK)])
        return carry

    lax.fori_loop(0, NCHUNK, chunk_body, 0)


def _gather_body(rows_hbm, table_hbm, out_hbm,
                 rows_a, rows_b, emb_a, emb_b, p_a, p_b,
                 sem_a, sem_b, sem_w):
    wid = lax.axis_index("s") * NC + lax.axis_index("c")
    lane = lax.broadcasted_iota(jnp.int32, (16,), 0)

    def fetch(k, rows_v, emb_v, sem):
        row_lo = wid * ROWS_PER_W + k * RCHUNK
        pltpu.sync_copy(rows_hbm.at[:, pl.ds(row_lo, RCHUNK)], rows_v)
        return [pltpu.async_copy(table_hbm.at[rows_v.at[l]], emb_v.at[l], sem)
                for l in range(L)]

    def produce(k, emb_v, p_v, cps):
        row_lo = wid * ROWS_PER_W + k * RCHUNK
        for cp in cps:
            cp.wait()

        def tr_body(l, carry2):
            lv = jnp.full((16,), l, jnp.int32)
            vals = [plsc.load_gather(emb_v, [lv, lane, jnp.full((16,), d, jnp.int32)])
                    for d in range(DIM)]
            for d in range(DIM):
                p_v[l, d, :] = vals[d]
            return carry2
        lax.fori_loop(0, L, tr_body, 0)
        return pltpu.async_copy(p_v, out_hbm.at[:, :, pl.ds(row_lo, RCHUNK)],
                                sem_w)

    def pair_body(kk, carry):
        k0 = 2 * kk
        k1 = k0 + 1
        cps_a = fetch(k0, rows_a, emb_a, sem_a)
        cps_b = fetch(k1, rows_b, emb_b, sem_b)
        wa = produce(k0, emb_a, p_a, cps_a)
        wb = produce(k1, emb_b, p_b, cps_b)
        wa.wait()
        wb.wait()
        return carry

    lax.fori_loop(0, NCHUNK // 2, pair_body, 0)


@jax.jit
def _lookup(raw_idx, keys2d, samp, table):
    mesh = plsc.VectorSubcoreMesh(core_axis_name="c", subcore_axis_name="s",
                                  num_cores=NC, num_subcores=NS)
    params = pltpu.CompilerParams(needs_layout_passes=False,
                                  use_tc_tiling_on_sc=False)
    k1 = pl.kernel(
        _search_body,
        out_type=jax.ShapeDtypeStruct((L, B), jnp.int32),
        mesh=mesh,
        compiler_params=params,
        scratch_types=[
            pltpu.VMEM((NPAD,), jnp.int32),
            pltpu.VMEM((RCHUNK, L), jnp.int32),
            pltpu.VMEM((L, RCHUNK), jnp.int32),
            pltpu.VMEM((L, RCHUNK, STRIDE), jnp.int32),
            pltpu.VMEM((L, RCHUNK), jnp.int32),
            pltpu.SemaphoreType.DMA,
        ],
    )
    rows_all = k1(raw_idx, keys2d, samp)
    k2 = pl.kernel(
        _gather_body,
        out_type=jax.ShapeDtypeStruct((L, DIM, B), jnp.float32),
        mesh=mesh,
        compiler_params=params,
        scratch_types=[
            pltpu.VMEM((L, RCHUNK), jnp.int32),
            pltpu.VMEM((L, RCHUNK), jnp.int32),
            pltpu.VMEM((L, RCHUNK, DIM), jnp.float32),
            pltpu.VMEM((L, RCHUNK, DIM), jnp.float32),
            pltpu.VMEM((L, DIM, RCHUNK), jnp.float32),
            pltpu.VMEM((L, DIM, RCHUNK), jnp.float32),
            pltpu.SemaphoreType.DMA,
            pltpu.SemaphoreType.DMA,
            pltpu.SemaphoreType.DMA,
        ],
    )
    return k2(rows_all, table)


def kernel(raw_idx, keys, table):
    keys2d = keys.reshape(NSAMP, STRIDE)
    samp = jnp.full((NPAD,), PAD_VAL, jnp.int32).at[:NSAMP].set(keys2d[:, 0])
    out_p = _lookup(raw_idx, keys2d, samp, table)
    return jnp.transpose(out_p, (2, 0, 1))
